# Initial kernel scaffold; baseline (speedup 1.0000x reference)
import jax
import jax.numpy as jnp
from jax.experimental import pallas as pl


def kernel(x, edge_index, batch, W_rel0, b_rel0, W_root0, W_rel1, b_rel1, W_root1, W_fc, b_fc):
    raise NotImplementedError("write your pallas kernel here")


# speedup vs baseline: 5.2027x; 5.2027x over previous
import functools

import jax
import jax.numpy as jnp
from jax import lax
from jax.experimental import pallas as pl
from jax.experimental.pallas import tpu as pltpu
from jax.experimental.pallas import tpu_sc as plsc

N = 10000
E = 320000
D = 128
OUT = 128

NC = 2
NS = 16
NW = NC * NS

CHUNK = 80
EDGES_PER_TILE = E // NW
CH_PER_TILE = EDGES_PER_TILE // CHUNK
ROWS_MAIN = 624
ROW0_LAST = ROWS_MAIN * (NS - 1)
ROWS_LAST = N - ROW0_LAST

_ZBUF_ROWS = 128


def _segsum_sc(h, src, dst):
    mesh = plsc.VectorSubcoreMesh(core_axis_name="c", subcore_axis_name="s",
                                  num_cores=NC, num_subcores=NS)

    @functools.partial(
        pl.kernel,
        mesh=mesh,
        out_type=[
            jax.ShapeDtypeStruct((N, D), jnp.float32),
            jax.ShapeDtypeStruct((N, D), jnp.float32),
        ],
        scratch_types=[
            pltpu.VMEM_SHARED((N, D), jnp.float32),
            pltpu.VMEM((CHUNK,), jnp.int32),
            pltpu.VMEM((CHUNK,), jnp.int32),
            pltpu.VMEM((CHUNK, D), jnp.float32),
            pltpu.VMEM((_ZBUF_ROWS, D), jnp.float32),
            pltpu.SemaphoreType.DMA,
        ],
    )
    def k(h_hbm, src_hbm, dst_hbm, out0_hbm, out1_hbm,
          acc, sidx, didx, rows, zbuf, sem):
        c = lax.axis_index("c")
        s = lax.axis_index("s")
        wid = c * NS + s

        def zb(i, _):
            r = i // (D // 16)
            col = (i % (D // 16)) * 16
            zbuf[r, pl.ds(col, 16)] = jnp.zeros((16,), jnp.float32)
            return 0
        lax.fori_loop(0, _ZBUF_ROWS * (D // 16), zb, 0)

        def zero_rows(row0, nrows):
            done = 0
            while done < nrows:
                nr = min(_ZBUF_ROWS, nrows - done)
                pltpu.sync_copy(zbuf.at[pl.ds(0, nr)],
                                acc.at[pl.ds(row0 + done, nr)])
                done += nr

        @pl.when(s < NS - 1)
        def _():
            zero_rows(s * ROWS_MAIN, ROWS_MAIN)

        @pl.when(s == NS - 1)
        def _():
            zero_rows(ROW0_LAST, ROWS_LAST)

        plsc.subcore_barrier()

        base0 = wid * EDGES_PER_TILE

        def it(i, _):
            base = base0 + i * CHUNK
            pltpu.sync_copy(src_hbm.at[pl.ds(base, CHUNK)], sidx)
            pltpu.sync_copy(dst_hbm.at[pl.ds(base, CHUNK)], didx)
            pltpu.async_copy(h_hbm.at[sidx], rows, sem).wait()
            pltpu.sync_copy(rows, acc.at[didx], add=True)
            return 0
        lax.fori_loop(0, CH_PER_TILE, it, 0)

        plsc.subcore_barrier()

        def flush(out_hbm):
            @pl.when(s < NS - 1)
            def _():
                pltpu.sync_copy(acc.at[pl.ds(s * ROWS_MAIN, ROWS_MAIN)],
                                out_hbm.at[pl.ds(s * ROWS_MAIN, ROWS_MAIN)])

            @pl.when(s == NS - 1)
            def _():
                pltpu.sync_copy(acc.at[pl.ds(ROW0_LAST, ROWS_LAST)],
                                out_hbm.at[pl.ds(ROW0_LAST, ROWS_LAST)])

        @pl.when(c == 0)
        def _():
            flush(out0_hbm)

        @pl.when(c == 1)
        def _():
            flush(out1_hbm)

    return k(h, src, dst)


_BR = 2000


def _dotT(a, w):
    return lax.dot_general(a, w, (((1,), (1,)), ((), ())),
                           preferred_element_type=jnp.float32)


def _lin1_body(p0, p1, x, wr, wt, b, o):
    agg = p0[...] + p1[...]
    o[...] = _dotT(agg, wr[...]) + _dotT(x[...], wt[...]) + b[...]


def _lin1(p0, p1, x, W_rel, W_root, b_rel):
    grid = (N // _BR,)
    row = pl.BlockSpec((_BR, D), lambda i: (i, 0))
    full = pl.BlockSpec((D, D), lambda i: (0, 0))
    bias = pl.BlockSpec((1, D), lambda i: (0, 0))
    return pl.pallas_call(
        _lin1_body,
        grid=grid,
        in_specs=[row, row, row, full, full, bias],
        out_specs=row,
        out_shape=jax.ShapeDtypeStruct((N, D), jnp.float32),
    )(p0, p1, x, W_rel, W_root, b_rel.reshape(1, D))


def _lin2_body(q0, q1, h, wfc, wr, wt, b1, bfc, o):
    g1 = jnp.dot(wfc[...], wr[...], preferred_element_type=jnp.float32)
    g2 = jnp.dot(wfc[...], wt[...], preferred_element_type=jnp.float32)
    agg = q0[...] + q1[...]
    cvec = _dotT(b1[...], wfc[...]) + bfc[...]
    o[...] = _dotT(agg, g1) + _dotT(h[...], g2) + cvec


def _lin2(q0, q1, h, W_fc, W_rel, W_root, b_rel, b_fc):
    grid = (N // _BR,)
    row = pl.BlockSpec((_BR, D), lambda i: (i, 0))
    full = pl.BlockSpec((D, D), lambda i: (0, 0))
    fc = pl.BlockSpec((OUT, D), lambda i: (0, 0))
    bias = pl.BlockSpec((1, D), lambda i: (0, 0))
    bias_o = pl.BlockSpec((1, OUT), lambda i: (0, 0))
    out_row = pl.BlockSpec((_BR, OUT), lambda i: (i, 0))
    return pl.pallas_call(
        _lin2_body,
        grid=grid,
        in_specs=[row, row, row, fc, full, full, bias, bias_o],
        out_specs=out_row,
        out_shape=jax.ShapeDtypeStruct((N, OUT), jnp.float32),
    )(q0, q1, h, W_fc, W_rel, W_root, b_rel.reshape(1, D), b_fc.reshape(1, OUT))


def kernel(x, edge_index, batch, W_rel0, b_rel0, W_root0,
           W_rel1, b_rel1, W_root1, W_fc, b_fc):
    src = edge_index[0]
    dst = edge_index[1]
    p0, p1 = _segsum_sc(x, src, dst)
    h1 = _lin1(p0, p1, x, W_rel0, W_root0, b_rel0)
    q0, q1 = _segsum_sc(h1, src, dst)
    return _lin2(q0, q1, h1, W_fc, W_rel1, W_root1, b_rel1, b_fc)
